# baseline (speedup 1.0000x reference)
import jax
import jax.numpy as jnp
from jax.experimental import pallas as pl
from jax.experimental.pallas import tpu as pltpu

_N = 8192
_BM = 256
_NBUF = 4
_NBLK = _N // _BM


def _gcn_kernel(x_ref, wt_ref, b_ref, adj_hbm, out_ref, bufs, support, sems):
    def start_copy(i):
        slot = i % _NBUF
        pltpu.make_async_copy(
            adj_hbm.at[pl.ds(i * _BM, _BM), :],
            bufs.at[slot],
            sems.at[slot],
        ).start()

    for i in range(_NBUF):
        start_copy(i)

    support[...] = (
        jnp.dot(x_ref[...], wt_ref[...], preferred_element_type=jnp.float32)
        + b_ref[...]
    )

    for i in range(_NBLK):
        slot = i % _NBUF
        pltpu.make_async_copy(
            adj_hbm.at[pl.ds(i * _BM, _BM), :],
            bufs.at[slot],
            sems.at[slot],
        ).wait()
        out_ref[pl.ds(i * _BM, _BM), :] = jnp.dot(
            bufs[slot], support[...], preferred_element_type=jnp.float32
        )
        if i + _NBUF < _NBLK:
            start_copy(i + _NBUF)


@jax.jit
def kernel(input, adj, W, b):
    n, d_in = input.shape
    d_out = W.shape[0]
    wt = W.T
    b2 = b.reshape(1, d_out)
    return pl.pallas_call(
        _gcn_kernel,
        in_specs=[
            pl.BlockSpec(memory_space=pltpu.MemorySpace.VMEM),
            pl.BlockSpec(memory_space=pltpu.MemorySpace.VMEM),
            pl.BlockSpec(memory_space=pltpu.MemorySpace.VMEM),
            pl.BlockSpec(memory_space=pltpu.MemorySpace.HBM),
        ],
        out_specs=pl.BlockSpec(memory_space=pltpu.MemorySpace.VMEM),
        out_shape=jax.ShapeDtypeStruct((n, d_out), jnp.float32),
        scratch_shapes=[
            pltpu.VMEM((_NBUF, _BM, n), jnp.float32),
            pltpu.VMEM((n, d_out), jnp.float32),
            pltpu.SemaphoreType.DMA((_NBUF,)),
        ],
    )(input, wt, b2, adj)


# speedup vs baseline: 1.0072x; 1.0072x over previous
import functools

import jax
import jax.numpy as jnp
from jax.experimental import pallas as pl
from jax.experimental.pallas import tpu as pltpu

_BM = 256


def _gcn_kernel(x_ref, wt_ref, b_ref, adj_a_ref, adj_b_ref, out_ref, support_ref):
    @pl.when(pl.program_id(0) == 0)
    def _compute_support():
        support_ref[...] = (
            jnp.dot(x_ref[...], wt_ref[...], preferred_element_type=jnp.float32)
            + b_ref[...]
        )

    s = support_ref[...]
    out_ref[0] = jnp.dot(adj_a_ref[0], s, preferred_element_type=jnp.float32)
    out_ref[1] = jnp.dot(adj_b_ref[0], s, preferred_element_type=jnp.float32)


@jax.jit
def kernel(input, adj, W, b):
    n, d_in = input.shape
    d_out = W.shape[0]
    wt = W.T
    b2 = b.reshape(1, d_out)
    half = n // 2
    adj3 = adj.reshape(2, half, n)
    grid = (half // _BM,)
    out = pl.pallas_call(
        _gcn_kernel,
        grid=grid,
        in_specs=[
            pl.BlockSpec((n, d_in), lambda i: (0, 0)),
            pl.BlockSpec((d_in, d_out), lambda i: (0, 0)),
            pl.BlockSpec((1, d_out), lambda i: (0, 0)),
            pl.BlockSpec((1, _BM, n), lambda i: (0, i, 0)),
            pl.BlockSpec((1, _BM, n), lambda i: (1, i, 0)),
        ],
        out_specs=pl.BlockSpec((2, _BM, d_out), lambda i: (0, i, 0)),
        out_shape=jax.ShapeDtypeStruct((2, half, d_out), jnp.float32),
        scratch_shapes=[pltpu.VMEM((n, d_out), jnp.float32)],
        compiler_params=pltpu.CompilerParams(
            dimension_semantics=("arbitrary",),
        ),
    )(input, wt, b2, adj3, adj3)
    return out.reshape(n, d_out)


# speedup vs baseline: 1.0848x; 1.0771x over previous
import functools

import jax
import jax.numpy as jnp
from jax.experimental import pallas as pl
from jax.experimental.pallas import tpu as pltpu

_BM = 256


def _gcn_kernel(x_ref, w_ref, b_ref, adj_ref, out_ref, support_ref):
    @pl.when(pl.program_id(0) == 0)
    def _compute_support():
        support_ref[...] = (
            jax.lax.dot_general(
                x_ref[...],
                w_ref[...],
                dimension_numbers=(((1,), (1,)), ((), ())),
                preferred_element_type=jnp.float32,
            )
            + b_ref[...]
        )

    out_ref[...] = jnp.dot(
        adj_ref[...], support_ref[...], preferred_element_type=jnp.float32
    )


@jax.jit
def kernel(input, adj, W, b):
    n, d_in = input.shape
    d_out = W.shape[0]
    b2 = b.reshape(1, d_out)
    grid = (n // _BM,)
    return pl.pallas_call(
        _gcn_kernel,
        grid=grid,
        in_specs=[
            pl.BlockSpec((n, d_in), lambda i: (0, 0)),
            pl.BlockSpec((d_out, d_in), lambda i: (0, 0)),
            pl.BlockSpec((1, d_out), lambda i: (0, 0)),
            pl.BlockSpec((_BM, n), lambda i: (i, 0)),
        ],
        out_specs=pl.BlockSpec((_BM, d_out), lambda i: (i, 0)),
        out_shape=jax.ShapeDtypeStruct((n, d_out), jnp.float32),
        scratch_shapes=[pltpu.VMEM((n, d_out), jnp.float32)],
        compiler_params=pltpu.CompilerParams(
            dimension_semantics=("arbitrary",),
        ),
    )(input, W, b2, adj)
